# baseline (speedup 1.0000x reference)
import jax
import jax.numpy as jnp
from jax import lax
from jax.experimental import pallas as pl

ROWS = 128
COLS = 32768
BR = 32

_BIG = 2**30


HALF = COLS // 2


def _tc_body(x0_ref, x1_ref, o_ref):
    i = pl.program_id(0)
    v0 = x0_ref[...]
    v1 = x1_ref[...]
    idx = lax.broadcasted_iota(jnp.int32, (BR, HALF), 1)
    m0 = jnp.max(v0, axis=1, keepdims=True)
    m1 = jnp.max(v1, axis=1, keepdims=True)
    m = jnp.maximum(m0, m1)
    cand0 = jnp.where(v0 == m, idx, _BIG)
    cand1 = jnp.where(v1 == m, idx + HALF, _BIG)
    res = jnp.minimum(jnp.min(cand0, axis=1), jnp.min(cand1, axis=1))

    dup = jnp.concatenate([res] * (ROWS // BR)).reshape(1, ROWS)
    lane = lax.broadcasted_iota(jnp.int32, (1, ROWS), 1)
    keep = (lane // BR) == i
    prev = o_ref[...].reshape(1, ROWS)
    o_ref[...] = jnp.where(keep, dup, prev).reshape(ROWS)


@jax.jit
def kernel(x):
    return pl.pallas_call(
        _tc_body,
        grid=(ROWS // BR,),
        in_specs=[pl.BlockSpec((BR, HALF), lambda i: (i, 0)),
                  pl.BlockSpec((BR, HALF), lambda i: (i, 1))],
        out_specs=pl.BlockSpec((ROWS,), lambda i: (0,)),
        out_shape=jax.ShapeDtypeStruct((ROWS,), jnp.int32),
    )(x, x)


# speedup vs baseline: 1.0182x; 1.0182x over previous
import jax
import jax.numpy as jnp
from jax import lax
from jax.experimental import pallas as pl

ROWS = 128
COLS = 32768
BR = 64

_BIG = 2**30


NSTREAM = 4
QUART = COLS // NSTREAM


def _tc_body(*refs):
    o_ref = refs[-1]
    xs = refs[:-1]
    i = pl.program_id(0)
    vs = [r[...] for r in xs]
    idx = lax.broadcasted_iota(jnp.int32, (BR, QUART), 1)
    m = vs[0].max(axis=1, keepdims=True)
    for v in vs[1:]:
        m = jnp.maximum(m, v.max(axis=1, keepdims=True))
    res = None
    for q, v in enumerate(vs):
        c = jnp.where(v == m, idx + q * QUART, _BIG).min(axis=1)
        res = c if res is None else jnp.minimum(res, c)

    dup = jnp.concatenate([res, res]).reshape(1, ROWS)
    lane = lax.broadcasted_iota(jnp.int32, (1, ROWS), 1)
    keep = (lane // BR) == i
    prev = o_ref[...].reshape(1, ROWS)
    o_ref[...] = jnp.where(keep, dup, prev).reshape(ROWS)


@jax.jit
def kernel(x):
    return pl.pallas_call(
        _tc_body,
        grid=(ROWS // BR,),
        in_specs=[pl.BlockSpec((BR, QUART), lambda i, q=q: (i, q))
                  for q in range(NSTREAM)],
        out_specs=pl.BlockSpec((ROWS,), lambda i: (0,)),
        out_shape=jax.ShapeDtypeStruct((ROWS,), jnp.int32),
    )(*([x] * NSTREAM))


# speedup vs baseline: 1.0710x; 1.0518x over previous
import jax
import jax.numpy as jnp
from jax import lax
from jax.experimental import pallas as pl

ROWS = 128
COLS = 32768
BR = 64

_BIG = 2**30


HALF = COLS // 2


def _tc_body(x0_ref, x1_ref, o_ref):
    i = pl.program_id(0)
    v0 = x0_ref[...]
    v1 = x1_ref[...]
    idx = lax.broadcasted_iota(jnp.int32, (BR, HALF), 1)
    m0 = jnp.max(v0, axis=1, keepdims=True)
    m1 = jnp.max(v1, axis=1, keepdims=True)
    m = jnp.maximum(m0, m1)
    cand0 = jnp.where(v0 == m, idx, _BIG)
    cand1 = jnp.where(v1 == m, idx + HALF, _BIG)
    res = jnp.minimum(jnp.min(cand0, axis=1), jnp.min(cand1, axis=1))

    dup = jnp.concatenate([res, res]).reshape(1, ROWS)
    lane = lax.broadcasted_iota(jnp.int32, (1, ROWS), 1)
    keep = (lane // BR) == i
    prev = o_ref[...].reshape(1, ROWS)
    o_ref[...] = jnp.where(keep, dup, prev).reshape(ROWS)


@jax.jit
def kernel(x):
    return pl.pallas_call(
        _tc_body,
        grid=(ROWS // BR,),
        in_specs=[pl.BlockSpec((BR, HALF), lambda i: (i, 0)),
                  pl.BlockSpec((BR, HALF), lambda i: (i, 1))],
        out_specs=pl.BlockSpec((ROWS,), lambda i: (0,)),
        out_shape=jax.ShapeDtypeStruct((ROWS,), jnp.int32),
    )(x, x)
